# baseline (speedup 1.0000x reference)
import jax
import jax.numpy as jnp
from jax.experimental import pallas as pl
from jax.experimental.pallas import tpu as pltpu

_BT = 512
_BO = 512


def _fused_kernel(perm_ref, w_ref, bias_ref, x_ref, out_ref, wd_ref, xbf_ref):
    i = pl.program_id(0)
    j = pl.program_id(1)
    k = x_ref.shape[1]
    bo = out_ref.shape[1]

    @pl.when(i == 0)
    def _build_w_block():
        perm = perm_ref[...]
        wv = w_ref[...].astype(jnp.bfloat16)
        row = jax.lax.broadcasted_iota(jnp.int16, (k, bo), 0)
        acc = jnp.zeros((k, bo), jnp.bfloat16)
        for s in range(perm.shape[0]):
            acc = acc + jnp.where(
                row == perm[s : s + 1, :], wv[s : s + 1, :], jnp.bfloat16(0.0)
            )
        wd_ref[:, pl.ds(j * bo, bo)] = acc

    @pl.when(j == 0)
    def _cast_x_block():
        xbf_ref[...] = x_ref[...].astype(jnp.bfloat16)

    out_ref[...] = (
        jnp.dot(
            xbf_ref[...],
            wd_ref[:, pl.ds(j * bo, bo)],
            preferred_element_type=jnp.float32,
        )
        + bias_ref[...]
    )


def kernel(x, permutations, weight, bias):
    lead = x.shape[:-1]
    k = x.shape[-1]
    t = 1
    for d in lead:
        t *= d
    x2 = x.reshape(t, k)
    o, s = permutations.shape
    perm_t = permutations.T.astype(jnp.int16)
    w_t = weight.T
    bias2 = bias.reshape(1, o)
    ni = t // _BT
    nj = o // _BO
    out = pl.pallas_call(
        _fused_kernel,
        grid=(ni, nj),
        in_specs=[
            pl.BlockSpec((s, _BO), lambda i, j: (0, j)),
            pl.BlockSpec((s, _BO), lambda i, j: (0, j)),
            pl.BlockSpec((1, _BO), lambda i, j: (0, j)),
            pl.BlockSpec((_BT, k), lambda i, j: (i, 0)),
        ],
        out_specs=pl.BlockSpec((_BT, _BO), lambda i, j: (i, j)),
        out_shape=jax.ShapeDtypeStruct((t, o), jnp.float32),
        scratch_shapes=[
            pltpu.VMEM((k, o), jnp.bfloat16),
            pltpu.VMEM((_BT, k), jnp.bfloat16),
        ],
    )(perm_t, w_t, bias2, x2)
    return out.reshape(*lead, o)


# speedup vs baseline: 1.0710x; 1.0710x over previous
import jax
import jax.numpy as jnp
from jax.experimental import pallas as pl
from jax.experimental.pallas import tpu as pltpu

_BO = 512


def _fused_kernel(perm_ref, w_ref, bias_ref, x_ref, out_ref, xbf_ref):
    k = x_ref.shape[1]
    bo = out_ref.shape[1]

    @pl.when(pl.program_id(0) == 0)
    def _cast_x():
        xbf_ref[...] = x_ref[...].astype(jnp.bfloat16)

    perm = perm_ref[...].astype(jnp.int16)
    wv = w_ref[...].astype(jnp.bfloat16)
    col = jax.lax.broadcasted_iota(jnp.int16, (bo, k), 1)
    acc = jnp.zeros((bo, k), jnp.bfloat16)
    for s in range(perm.shape[1]):
        acc = acc + jnp.where(
            col == perm[:, s : s + 1], wv[:, s : s + 1], jnp.bfloat16(0.0)
        )
    out_ref[...] = (
        jax.lax.dot_general(
            xbf_ref[...],
            acc,
            (((1,), (1,)), ((), ())),
            preferred_element_type=jnp.float32,
        )
        + bias_ref[...]
    )


def kernel(x, permutations, weight, bias):
    lead = x.shape[:-1]
    k = x.shape[-1]
    t = 1
    for d in lead:
        t *= d
    x2 = x.reshape(t, k)
    o, s = permutations.shape
    bias2 = bias.reshape(1, o)
    nj = o // _BO
    out = pl.pallas_call(
        _fused_kernel,
        grid=(nj,),
        in_specs=[
            pl.BlockSpec((_BO, s), lambda j: (j, 0)),
            pl.BlockSpec((_BO, s), lambda j: (j, 0)),
            pl.BlockSpec((1, _BO), lambda j: (0, j)),
            pl.BlockSpec((t, k), lambda j: (0, 0)),
        ],
        out_specs=pl.BlockSpec((t, _BO), lambda j: (0, j)),
        out_shape=jax.ShapeDtypeStruct((t, o), jnp.float32),
        scratch_shapes=[pltpu.VMEM((t, k), jnp.bfloat16)],
    )(permutations, weight, bias2, x2)
    return out.reshape(*lead, o)


# speedup vs baseline: 1.2835x; 1.1985x over previous
import jax
import jax.numpy as jnp
from jax.experimental import pallas as pl
from jax.experimental.pallas import tpu as pltpu

_BO = 512


_RC = 64


def _fused_kernel(perm_ref, w_ref, bias_ref, x_ref, out_ref, xbf_ref, wd_ref):
    k = x_ref.shape[1]
    bo = out_ref.shape[1]

    @pl.when(pl.program_id(0) == 0)
    def _cast_x():
        xbf_ref[...] = x_ref[...].astype(jnp.bfloat16)

    perm = perm_ref[...]
    wv = w_ref[...].astype(jnp.bfloat16)
    base = jax.lax.broadcasted_iota(jnp.int16, (_RC, bo), 0)
    for c in range(k // _RC):
        row = base + jnp.int16(c * _RC)
        acc = jnp.where(row == perm[0:1, :], wv[0:1, :], jnp.bfloat16(0.0))
        for s in range(1, perm.shape[0]):
            acc = acc + jnp.where(
                row == perm[s : s + 1, :], wv[s : s + 1, :], jnp.bfloat16(0.0)
            )
        wd_ref[pl.ds(c * _RC, _RC), :] = acc
    out_ref[...] = (
        jnp.dot(xbf_ref[...], wd_ref[...], preferred_element_type=jnp.float32)
        + bias_ref[...]
    )


def kernel(x, permutations, weight, bias):
    lead = x.shape[:-1]
    k = x.shape[-1]
    t = 1
    for d in lead:
        t *= d
    x2 = x.reshape(t, k)
    o, s = permutations.shape
    perm_t = permutations.T.astype(jnp.int16)
    w_t = weight.T
    bias2 = bias.reshape(1, o)
    nj = o // _BO
    out = pl.pallas_call(
        _fused_kernel,
        grid=(nj,),
        in_specs=[
            pl.BlockSpec((s, _BO), lambda j: (0, j)),
            pl.BlockSpec((s, _BO), lambda j: (0, j)),
            pl.BlockSpec((1, _BO), lambda j: (0, j)),
            pl.BlockSpec((t, k), lambda j: (0, 0)),
        ],
        out_specs=pl.BlockSpec((t, _BO), lambda j: (0, j)),
        out_shape=jax.ShapeDtypeStruct((t, o), jnp.float32),
        scratch_shapes=[
            pltpu.VMEM((t, k), jnp.bfloat16),
            pltpu.VMEM((k, _BO), jnp.bfloat16),
        ],
    )(perm_t, w_t, bias2, x2)
    return out.reshape(*lead, o)
